# baseline (speedup 1.0000x reference)
import functools

import jax
import jax.numpy as jnp
from jax import lax
from jax.experimental import pallas as pl
from jax.experimental.pallas import tpu as pltpu
from jax.experimental.pallas import tpu_sc as plsc

B = 16384
GMF_D = 8
MLP_D = 16
CHUNK = 128
N_CHUNKS = B // CHUNK
GMF_PACK = 128 // GMF_D
MLP_PACK = 128 // MLP_D
BLK = 1024


def _gather_sc(idx_all, gu_t, gi_t, mu_t, mi_t):
    info = plsc.get_sparse_core_info()
    NW = info.num_cores * info.num_subcores
    chunks_per_w = N_CHUNKS // NW

    mesh = plsc.VectorSubcoreMesh(core_axis_name="c", subcore_axis_name="s")

    @functools.partial(
        pl.kernel,
        mesh=mesh,
        compiler_params=pltpu.CompilerParams(use_tc_tiling_on_sc=True),
        out_type=[jax.ShapeDtypeStruct((B, 128), jnp.float32) for _ in range(4)],
        scratch_types=[
            pltpu.VMEM((4, CHUNK), jnp.int32),
            pltpu.VMEM((CHUNK, 128), jnp.float32),
            pltpu.VMEM((CHUNK, 128), jnp.float32),
            pltpu.VMEM((CHUNK, 128), jnp.float32),
            pltpu.VMEM((CHUNK, 128), jnp.float32),
            pltpu.SemaphoreType.DMA,
        ],
    )
    def gather_kernel(idx_hbm, gu_tab, gi_tab, mu_tab, mi_tab,
                      gu_out, gi_out, mu_out, mi_out,
                      sidx, b0, b1, b2, b3, sem):
        wid = lax.axis_index("s") * info.num_cores + lax.axis_index("c")
        tabs = (gu_tab, gi_tab, mu_tab, mi_tab)
        bufs = (b0, b1, b2, b3)
        outs = (gu_out, gi_out, mu_out, mi_out)
        for r in range(chunks_per_w):
            chunk = wid + r * NW
            for t in range(4):
                pltpu.sync_copy(idx_hbm.at[t, chunk], sidx.at[t])
            copies = [
                pltpu.async_copy(tabs[t].at[sidx.at[t]], bufs[t], sem)
                for t in range(4)
            ]
            for c in copies:
                c.wait()
            for t in range(4):
                pltpu.sync_copy(bufs[t], outs[t].at[pl.ds(chunk * CHUNK, CHUNK)])

    return gather_kernel(idx_all, gu_t, gi_t, mu_t, mi_t)


def _head_tc_body(u, it, gu128, gi128, mu128, mi128,
                  w1u128, w1i128, b1, w2, b2, wlg, wlh, bl, out):
    lane = lax.broadcasted_iota(jnp.int32, (BLK, 128), 1)
    gsel = lane // GMF_D
    gmask_u = (u[...] % GMF_PACK) == gsel
    gmask_i = (it[...] % GMF_PACK) == gsel
    row8 = lax.broadcasted_iota(jnp.int32, (128, GMF_D), 0)
    col8 = lax.broadcasted_iota(jnp.int32, (128, GMF_D), 1)
    s8 = ((row8 % GMF_D) == col8).astype(jnp.float32)
    gu = jnp.where(gmask_u, gu128[...], 0.0) @ s8
    gi = jnp.where(gmask_i, gi128[...], 0.0) @ s8
    gmf = gu * gi

    msel = lane // MLP_D
    mmask_u = (u[...] % MLP_PACK) == msel
    mmask_i = (it[...] % MLP_PACK) == msel
    h = (jnp.where(mmask_u, mu128[...], 0.0) @ w1u128[...]
         + jnp.where(mmask_i, mi128[...], 0.0) @ w1i128[...] + b1[...])
    h = jnp.maximum(h, 0.0)
    h = h @ w2[...] + b2[...]
    h = jnp.maximum(h, 0.0)
    logits = gmf @ wlg[...] + h @ wlh[...] + bl[...]
    out[...] = jax.nn.sigmoid(logits)


def kernel(user, item, gmf_user_emb, gmf_item_emb, mlp_user_emb, mlp_item_emb,
           W1, b1, W2, b2, Wl, bl):
    u32 = user.astype(jnp.int32)
    i32 = item.astype(jnp.int32)
    idx_all = jnp.stack([
        (u32 // GMF_PACK).reshape(N_CHUNKS, CHUNK),
        (i32 // GMF_PACK).reshape(N_CHUNKS, CHUNK),
        (u32 // MLP_PACK).reshape(N_CHUNKS, CHUNK),
        (i32 // MLP_PACK).reshape(N_CHUNKS, CHUNK),
    ])
    one = lax.optimization_barrier(jnp.float32(1.0))
    gu_t = (gmf_user_emb * one).reshape(-1, 128)
    gi_t = (gmf_item_emb * one).reshape(-1, 128)
    mu_t = (mlp_user_emb * one).reshape(-1, 128)
    mi_t = (mlp_item_emb * one).reshape(-1, 128)

    gu128, gi128, mu128, mi128 = _gather_sc(idx_all, gu_t, gi_t, mu_t, mi_t)

    w1u128 = jnp.tile(W1[:MLP_D], (MLP_PACK, 1))
    w1i128 = jnp.tile(W1[MLP_D:], (MLP_PACK, 1))
    wlg = Wl[:GMF_D]
    wlh = Wl[GMF_D:]
    b1r = b1.reshape(1, -1)
    b2r = b2.reshape(1, -1)
    blr = bl.reshape(1, 1)
    u2d = u32.reshape(B, 1)
    i2d = i32.reshape(B, 1)

    n_blk = B // BLK
    out = pl.pallas_call(
        _head_tc_body,
        grid=(n_blk,),
        in_specs=[
            pl.BlockSpec((BLK, 1), lambda i: (i, 0)),
            pl.BlockSpec((BLK, 1), lambda i: (i, 0)),
            pl.BlockSpec((BLK, 128), lambda i: (i, 0)),
            pl.BlockSpec((BLK, 128), lambda i: (i, 0)),
            pl.BlockSpec((BLK, 128), lambda i: (i, 0)),
            pl.BlockSpec((BLK, 128), lambda i: (i, 0)),
            pl.BlockSpec((128, MLP_D), lambda i: (0, 0)),
            pl.BlockSpec((128, MLP_D), lambda i: (0, 0)),
            pl.BlockSpec((1, MLP_D), lambda i: (0, 0)),
            pl.BlockSpec((MLP_D, GMF_D), lambda i: (0, 0)),
            pl.BlockSpec((1, GMF_D), lambda i: (0, 0)),
            pl.BlockSpec((GMF_D, 1), lambda i: (0, 0)),
            pl.BlockSpec((GMF_D, 1), lambda i: (0, 0)),
            pl.BlockSpec((1, 1), lambda i: (0, 0)),
        ],
        out_specs=pl.BlockSpec((BLK, 1), lambda i: (i, 0)),
        out_shape=jax.ShapeDtypeStruct((B, 1), jnp.float32),
    )(u2d, i2d, gu128, gi128, mu128, mi128,
      w1u128, w1i128, b1r, W2, b2r, wlg, wlh, blr)
    return out.reshape(-1)


# speedup vs baseline: 1.5775x; 1.5775x over previous
import functools

import jax
import jax.numpy as jnp
from jax import lax
from jax.experimental import pallas as pl
from jax.experimental.pallas import tpu as pltpu
from jax.experimental.pallas import tpu_sc as plsc

B = 16384
GMF_D = 8
MLP_D = 16
CHUNK = 128
N_CHUNKS = B // CHUNK
BLK = 1024


def _gather_sc(idx_all, gu_t, gi_t, mu_t, mi_t):
    info = plsc.get_sparse_core_info()
    NW = info.num_cores * info.num_subcores
    chunks_per_w = N_CHUNKS // NW

    mesh = plsc.VectorSubcoreMesh(core_axis_name="c", subcore_axis_name="s")

    @functools.partial(
        pl.kernel,
        mesh=mesh,
        out_type=[jax.ShapeDtypeStruct((B, MLP_D), jnp.float32)] * 4,
        scratch_types=[
            pltpu.VMEM((4, CHUNK), jnp.int32),
            pltpu.VMEM((CHUNK, MLP_D), jnp.float32),
            pltpu.VMEM((CHUNK, MLP_D), jnp.float32),
            pltpu.VMEM((CHUNK, MLP_D), jnp.float32),
            pltpu.VMEM((CHUNK, MLP_D), jnp.float32),
            pltpu.SemaphoreType.DMA,
        ],
        compiler_params=pltpu.CompilerParams(use_tc_tiling_on_sc=False),
    )
    def gather_kernel(idx_hbm, gu_tab, gi_tab, mu_tab, mi_tab,
                      gu_out, gi_out, mu_out, mi_out,
                      sidx, b0, b1, b2, b3, sem):
        wid = lax.axis_index("s") * info.num_cores + lax.axis_index("c")
        tabs = (gu_tab, gi_tab, mu_tab, mi_tab)
        bufs = (b0, b1, b2, b3)
        outs = (gu_out, gi_out, mu_out, mi_out)
        for r in range(chunks_per_w):
            chunk = wid + r * NW
            for t in range(4):
                pltpu.sync_copy(idx_hbm.at[t, chunk], sidx.at[t])
            copies = [
                pltpu.async_copy(tabs[t].at[sidx.at[t]], bufs[t], sem)
                for t in range(4)
            ]
            for c in copies:
                c.wait()
            for t in range(4):
                pltpu.sync_copy(bufs[t], outs[t].at[pl.ds(chunk * CHUNK, CHUNK)])

    return gather_kernel(idx_all, gu_t, gi_t, mu_t, mi_t)


def _head_tc_body(u, it, gu16, gi16, mu, mi,
                  w1u, w1i, b1, w2, b2, wlg, wlh, bl, out):
    u_even = (u[...] % 2) == 0
    i_even = (it[...] % 2) == 0
    gu = jnp.where(u_even, gu16[:, :GMF_D], gu16[:, GMF_D:])
    gi = jnp.where(i_even, gi16[:, :GMF_D], gi16[:, GMF_D:])
    gmf = gu * gi

    h = mu[...] @ w1u[...] + mi[...] @ w1i[...] + b1[...]
    h = jnp.maximum(h, 0.0)
    h = h @ w2[...] + b2[...]
    h = jnp.maximum(h, 0.0)
    logits = gmf @ wlg[...] + h @ wlh[...] + bl[...]
    out[...] = jax.nn.sigmoid(logits)


def kernel(user, item, gmf_user_emb, gmf_item_emb, mlp_user_emb, mlp_item_emb,
           W1, b1, W2, b2, Wl, bl):
    u32 = user.astype(jnp.int32)
    i32 = item.astype(jnp.int32)
    idx_all = jnp.stack([
        (u32 // 2).reshape(N_CHUNKS, CHUNK),
        (i32 // 2).reshape(N_CHUNKS, CHUNK),
        u32.reshape(N_CHUNKS, CHUNK),
        i32.reshape(N_CHUNKS, CHUNK),
    ])
    gu_t = gmf_user_emb.reshape(-1, 2 * GMF_D)
    gi_t = gmf_item_emb.reshape(-1, 2 * GMF_D)

    gu16, gi16, mu, mi = _gather_sc(idx_all, gu_t, gi_t,
                                    mlp_user_emb, mlp_item_emb)

    w1u = W1[:MLP_D]
    w1i = W1[MLP_D:]
    wlg = Wl[:GMF_D]
    wlh = Wl[GMF_D:]
    b1r = b1.reshape(1, -1)
    b2r = b2.reshape(1, -1)
    blr = bl.reshape(1, 1)
    u2d = u32.reshape(B, 1)
    i2d = i32.reshape(B, 1)

    n_blk = B // BLK
    out = pl.pallas_call(
        _head_tc_body,
        grid=(n_blk,),
        in_specs=[
            pl.BlockSpec((BLK, 1), lambda i: (i, 0)),
            pl.BlockSpec((BLK, 1), lambda i: (i, 0)),
            pl.BlockSpec((BLK, MLP_D), lambda i: (i, 0)),
            pl.BlockSpec((BLK, MLP_D), lambda i: (i, 0)),
            pl.BlockSpec((BLK, MLP_D), lambda i: (i, 0)),
            pl.BlockSpec((BLK, MLP_D), lambda i: (i, 0)),
            pl.BlockSpec((MLP_D, MLP_D), lambda i: (0, 0)),
            pl.BlockSpec((MLP_D, MLP_D), lambda i: (0, 0)),
            pl.BlockSpec((1, MLP_D), lambda i: (0, 0)),
            pl.BlockSpec((MLP_D, GMF_D), lambda i: (0, 0)),
            pl.BlockSpec((1, GMF_D), lambda i: (0, 0)),
            pl.BlockSpec((GMF_D, 1), lambda i: (0, 0)),
            pl.BlockSpec((GMF_D, 1), lambda i: (0, 0)),
            pl.BlockSpec((1, 1), lambda i: (0, 0)),
        ],
        out_specs=pl.BlockSpec((BLK, 1), lambda i: (i, 0)),
        out_shape=jax.ShapeDtypeStruct((B, 1), jnp.float32),
    )(u2d, i2d, gu16, gi16, mu, mi,
      w1u, w1i, b1r, W2, b2r, wlg, wlh, blr)
    return out.reshape(-1)


# speedup vs baseline: 1.5806x; 1.0020x over previous
import functools

import jax
import jax.numpy as jnp
from jax import lax
from jax.experimental import pallas as pl
from jax.experimental.pallas import tpu as pltpu
from jax.experimental.pallas import tpu_sc as plsc

B = 16384
GMF_D = 8
MLP_D = 16
CHUNK = 128
N_CHUNKS = B // CHUNK
BLK = 1024


def _gather_sc(idx_all, gu_t, gi_t, mu_t, mi_t):
    info = plsc.get_sparse_core_info()
    NW = info.num_cores * info.num_subcores
    BW = B // NW

    mesh = plsc.VectorSubcoreMesh(core_axis_name="c", subcore_axis_name="s")

    @functools.partial(
        pl.kernel,
        mesh=mesh,
        out_type=[jax.ShapeDtypeStruct((B, MLP_D), jnp.float32)] * 4,
        scratch_types=[
            pltpu.VMEM((4, BW), jnp.int32),
            pltpu.VMEM((BW, MLP_D), jnp.float32),
            pltpu.VMEM((BW, MLP_D), jnp.float32),
            pltpu.VMEM((BW, MLP_D), jnp.float32),
            pltpu.VMEM((BW, MLP_D), jnp.float32),
            pltpu.SemaphoreType.DMA,
        ],
        compiler_params=pltpu.CompilerParams(use_tc_tiling_on_sc=False),
    )
    def gather_kernel(idx_hbm, gu_tab, gi_tab, mu_tab, mi_tab,
                      gu_out, gi_out, mu_out, mi_out,
                      sidx, b0, b1, b2, b3, sem):
        wid = lax.axis_index("s") * info.num_cores + lax.axis_index("c")
        tabs = (gu_tab, gi_tab, mu_tab, mi_tab)
        bufs = (b0, b1, b2, b3)
        outs = (gu_out, gi_out, mu_out, mi_out)
        pltpu.sync_copy(idx_hbm.at[wid], sidx)
        copies = [
            pltpu.async_copy(tabs[t].at[sidx.at[t]], bufs[t], sem)
            for t in range(4)
        ]
        for c in copies:
            c.wait()
        for t in range(4):
            pltpu.sync_copy(bufs[t], outs[t].at[pl.ds(wid * BW, BW)])

    return gather_kernel(idx_all, gu_t, gi_t, mu_t, mi_t)


def _head_tc_body(u, it, gu16, gi16, mu, mi,
                  w1u, w1i, b1, w2, b2, wlg, wlh, bl, out):
    u_even = (u[...] % 2) == 0
    i_even = (it[...] % 2) == 0
    gu = jnp.where(u_even, gu16[:, :GMF_D], gu16[:, GMF_D:])
    gi = jnp.where(i_even, gi16[:, :GMF_D], gi16[:, GMF_D:])
    gmf = gu * gi

    h = mu[...] @ w1u[...] + mi[...] @ w1i[...] + b1[...]
    h = jnp.maximum(h, 0.0)
    h = h @ w2[...] + b2[...]
    h = jnp.maximum(h, 0.0)
    logits = gmf @ wlg[...] + h @ wlh[...] + bl[...]
    out[...] = jax.nn.sigmoid(logits)


def kernel(user, item, gmf_user_emb, gmf_item_emb, mlp_user_emb, mlp_item_emb,
           W1, b1, W2, b2, Wl, bl):
    u32 = user.astype(jnp.int32)
    i32 = item.astype(jnp.int32)
    nw = B // 512
    idx_all = jnp.stack([
        (u32 // 2).reshape(nw, 512),
        (i32 // 2).reshape(nw, 512),
        u32.reshape(nw, 512),
        i32.reshape(nw, 512),
    ], axis=1)
    gu_t = gmf_user_emb.reshape(-1, 2 * GMF_D)
    gi_t = gmf_item_emb.reshape(-1, 2 * GMF_D)

    gu16, gi16, mu, mi = _gather_sc(idx_all, gu_t, gi_t,
                                    mlp_user_emb, mlp_item_emb)

    w1u = W1[:MLP_D]
    w1i = W1[MLP_D:]
    wlg = Wl[:GMF_D]
    wlh = Wl[GMF_D:]
    b1r = b1.reshape(1, -1)
    b2r = b2.reshape(1, -1)
    blr = bl.reshape(1, 1)
    u2d = u32.reshape(B, 1)
    i2d = i32.reshape(B, 1)

    n_blk = B // BLK
    out = pl.pallas_call(
        _head_tc_body,
        grid=(n_blk,),
        in_specs=[
            pl.BlockSpec((BLK, 1), lambda i: (i, 0)),
            pl.BlockSpec((BLK, 1), lambda i: (i, 0)),
            pl.BlockSpec((BLK, MLP_D), lambda i: (i, 0)),
            pl.BlockSpec((BLK, MLP_D), lambda i: (i, 0)),
            pl.BlockSpec((BLK, MLP_D), lambda i: (i, 0)),
            pl.BlockSpec((BLK, MLP_D), lambda i: (i, 0)),
            pl.BlockSpec((MLP_D, MLP_D), lambda i: (0, 0)),
            pl.BlockSpec((MLP_D, MLP_D), lambda i: (0, 0)),
            pl.BlockSpec((1, MLP_D), lambda i: (0, 0)),
            pl.BlockSpec((MLP_D, GMF_D), lambda i: (0, 0)),
            pl.BlockSpec((1, GMF_D), lambda i: (0, 0)),
            pl.BlockSpec((GMF_D, 1), lambda i: (0, 0)),
            pl.BlockSpec((GMF_D, 1), lambda i: (0, 0)),
            pl.BlockSpec((1, 1), lambda i: (0, 0)),
        ],
        out_specs=pl.BlockSpec((BLK, 1), lambda i: (i, 0)),
        out_shape=jax.ShapeDtypeStruct((B, 1), jnp.float32),
    )(u2d, i2d, gu16, gi16, mu, mi,
      w1u, w1i, b1r, W2, b2r, wlg, wlh, blr)
    return out.reshape(-1)
